# baseline (speedup 1.0000x reference)
import functools

import jax
import jax.numpy as jnp
from jax import lax
from jax.experimental import pallas as pl
from jax.experimental.pallas import tpu as pltpu
from jax.experimental.pallas import tpu_sc as plsc

VOCAB = 76
DIM = 768
B, L = 64, 1024

NC, NS = 2, 16
NW = NC * NS
ROWS = B * L
ROWS_PER_W = ROWS // NW
CHUNK = 16
NCHUNK = ROWS_PER_W // CHUNK
NBUF = 4
NGROUP = NCHUNK // NBUF
VPAD = 80
LANES = 16


def _sc_lookup(ids_hbm, table_hbm, out_hbm, idx_v, table_v, bufs, so):
    wid = lax.axis_index("s") * NC + lax.axis_index("c")
    base = wid * ROWS_PER_W * DIM
    pltpu.sync_copy(table_hbm, table_v)
    pltpu.sync_copy(ids_hbm.at[wid], idx_v)

    colofs = [lax.iota(jnp.int32, LANES) + k * LANES for k in range(DIM // LANES)]

    def writeout(c, b):
        return pltpu.make_async_copy(
            bufs[b], out_hbm.at[pl.ds(base + c * (CHUNK * DIM), CHUNK * DIM)], so[b]
        )

    def group_body(j, carry):
        for b in range(NBUF):
            c = j * NBUF + b

            @pl.when(c >= NBUF)
            def _():
                writeout(c - NBUF, b).wait()

            ids16 = idx_v[pl.ds(c * CHUNK, CHUNK)]
            rowbase = ids16 * DIM

            @plsc.parallel_loop(0, CHUNK, 1, unroll=2)
            def row_body(r, b=b):
                rb = rowbase.at[jnp.full((LANES,), r, jnp.int32)].get(
                    mode=lax.GatherScatterMode.PROMISE_IN_BOUNDS
                )
                for k in range(DIM // LANES):
                    val = plsc.load_gather(table_v, [rb + colofs[k]])
                    bufs[b][pl.ds(r * DIM + k * LANES, LANES)] = val

            writeout(c, b).start()

        return carry

    lax.fori_loop(0, NGROUP, group_body, 0)
    for b in range(NBUF):
        writeout(NCHUNK - NBUF + b, b).wait()


@functools.partial(jax.jit, static_argnames=())
def kernel(phoneme_ids, table):
    ids = phoneme_ids.astype(jnp.int32).reshape(NW, ROWS_PER_W)
    table_pad = (
        jnp.zeros((VPAD, DIM), jnp.float32).at[:VOCAB].set(table).reshape(VPAD * DIM)
    )
    mesh = plsc.VectorSubcoreMesh(
        core_axis_name="c", subcore_axis_name="s", num_cores=NC, num_subcores=NS
    )
    out = pl.kernel(
        _sc_lookup,
        out_type=jax.ShapeDtypeStruct((ROWS * DIM,), jnp.float32),
        mesh=mesh,
        compiler_params=pltpu.CompilerParams(needs_layout_passes=False),
        scratch_types=[
            pltpu.VMEM((ROWS_PER_W,), jnp.int32),
            pltpu.VMEM((VPAD * DIM,), jnp.float32),
            [pltpu.VMEM((CHUNK * DIM,), jnp.float32) for _ in range(NBUF)],
            [pltpu.SemaphoreType.DMA for _ in range(NBUF)],
        ],
    )(ids, table_pad)
    embeds = out.reshape(B, L, DIM)
    mask = jnp.ones((B, L), dtype=jnp.float32)
    return (embeds, mask)


# speedup vs baseline: 1.1198x; 1.1198x over previous
import functools

import jax
import jax.numpy as jnp
from jax import lax
from jax.experimental import pallas as pl
from jax.experimental.pallas import tpu as pltpu
from jax.experimental.pallas import tpu_sc as plsc

VOCAB = 76
DIM = 768
B, L = 64, 1024

NC, NS = 2, 16
NW = NC * NS
ROWS = B * L
ROWS_PER_W = ROWS // NW
CHUNK = 16
NCHUNK = ROWS_PER_W // CHUNK
NBUF = 4
NGROUP = NCHUNK // NBUF
VPAD = 80
LANES = 16


def _sc_lookup(ids_hbm, table_hbm, out_hbm, idx_v, table_v, bufs, so):
    wid = lax.axis_index("s") * NC + lax.axis_index("c")
    base = wid * ROWS_PER_W * DIM
    pltpu.sync_copy(table_hbm, table_v)
    pltpu.sync_copy(ids_hbm.at[wid], idx_v)

    lane_iota = lax.iota(jnp.int32, LANES)

    def writeout(c, b):
        return pltpu.make_async_copy(
            bufs[b], out_hbm.at[pl.ds(base + c * (CHUNK * DIM), CHUNK * DIM)], so[b]
        )

    def group_body(j, carry):
        for b in range(NBUF):
            c = j * NBUF + b

            @pl.when(c >= NBUF)
            def _():
                writeout(c - NBUF, b).wait()

            ids16 = idx_v[pl.ds(c * CHUNK, CHUNK)]
            rowbase = ids16 * DIM

            @plsc.parallel_loop(0, CHUNK, 1, unroll=2)
            def row_body(r, b=b):
                rb = jnp.sum(jnp.where(lane_iota == r, rowbase, 0))
                for k in range(DIM // LANES):
                    val = table_v[pl.ds(rb + k * LANES, LANES)]
                    bufs[b][pl.ds(r * DIM + k * LANES, LANES)] = val

            writeout(c, b).start()

        return carry

    lax.fori_loop(0, NGROUP, group_body, 0)
    for b in range(NBUF):
        writeout(NCHUNK - NBUF + b, b).wait()


@functools.partial(jax.jit, static_argnames=())
def kernel(phoneme_ids, table):
    ids = phoneme_ids.astype(jnp.int32).reshape(NW, ROWS_PER_W)
    table_pad = (
        jnp.zeros((VPAD, DIM), jnp.float32).at[:VOCAB].set(table).reshape(VPAD * DIM)
    )
    mesh = plsc.VectorSubcoreMesh(
        core_axis_name="c", subcore_axis_name="s", num_cores=NC, num_subcores=NS
    )
    out = pl.kernel(
        _sc_lookup,
        out_type=jax.ShapeDtypeStruct((ROWS * DIM,), jnp.float32),
        mesh=mesh,
        compiler_params=pltpu.CompilerParams(needs_layout_passes=False),
        scratch_types=[
            pltpu.VMEM((ROWS_PER_W,), jnp.int32),
            pltpu.VMEM((VPAD * DIM,), jnp.float32),
            [pltpu.VMEM((CHUNK * DIM,), jnp.float32) for _ in range(NBUF)],
            [pltpu.SemaphoreType.DMA for _ in range(NBUF)],
        ],
    )(ids, table_pad)
    embeds = out.reshape(B, L, DIM)
    mask = jnp.ones((B, L), dtype=jnp.float32)
    return (embeds, mask)
